# baseline (speedup 1.0000x reference)
import functools

import jax
import jax.numpy as jnp
from jax.experimental import pallas as pl
from jax.experimental.pallas import tpu as pltpu

_X_DIM = 512
_DELTA_DIM = 512
_LAYER_N = 2
_EPS = 1e-5
_TL = 1024
_K = 4


def _gt_kernel(x_dim, eps, L, TL, K, n_edits, n_steps,
               x_hbm, len_ref,
               xn_hbm, dw_ref, db_ref,
               x_buf, xn_buf, in_sems, out_sems):
    i = pl.program_id(0)
    b = jax.lax.rem(i, 2)
    CH = TL // K

    def in_copy(step, slot, k):
        return pltpu.make_async_copy(
            x_hbm.at[pl.ds(step * TL + k * CH, CH), :],
            x_buf.at[slot, pl.ds(k * CH, CH), :],
            in_sems.at[slot, k])

    def out_copy(step, slot, k):
        return pltpu.make_async_copy(
            xn_buf.at[slot, pl.ds(k * CH, CH), :],
            xn_hbm.at[pl.ds(step * TL + k * CH, CH), :],
            out_sems.at[slot, k])

    @pl.when(i == 0)
    def _():
        for k in range(K):
            in_copy(0, 0, k).start()
        dw_ref[...] = jnp.zeros_like(dw_ref)
        db_ref[...] = jnp.zeros_like(db_ref)

    @pl.when(i + 1 < n_steps)
    def _():
        for k in range(K):
            in_copy(i + 1, 1 - b, k).start()

    for k in range(K):
        in_copy(i, b, k).wait()

    @pl.when(i >= 2)
    def _():
        for k in range(K):
            out_copy(i - 2, b, k).wait()

    x = x_buf[b]
    D = x.shape[-1]
    inv_d = jnp.float32(1.0 / D)

    s1 = jnp.sum(x, axis=-1, keepdims=True)
    s2 = jnp.sum(x * x, axis=-1, keepdims=True)
    mean = s1 * inv_d
    var = jnp.maximum(s2 * inv_d - mean * mean, 0.0)
    xn = (x - mean) * jax.lax.rsqrt(var + eps)
    xn_buf[b] = xn

    for k in range(K):
        out_copy(i, b, k).start()

    lane_ids = i * TL + jax.lax.broadcasted_iota(jnp.int32, (1, TL), 1)
    scale_lane = jnp.zeros((1, TL), jnp.float32)
    lo = jnp.int32(0)
    for e in range(n_edits):
        ln = len_ref[e]
        hi = lo + ln
        pe = 1.0 / ln.astype(jnp.float32)
        scale_lane = jnp.where((lane_ids >= lo) & (lane_ids < hi),
                               pe, scale_lane)
        lo = hi
    scale = jnp.transpose(scale_lane, (1, 0))

    x2f = xn[:, x_dim:] * scale
    db_ref[...] += jnp.sum(x2f, axis=0, keepdims=True)

    x1b = xn[:, :x_dim].astype(jnp.bfloat16)
    x2b = x2f.astype(jnp.bfloat16)
    dw_ref[...] += jax.lax.dot_general(
        x1b, x2b,
        dimension_numbers=(((0,), (0,)), ((), ())),
        preferred_element_type=jnp.float32)

    @pl.when(i == n_steps - 1)
    def _():
        for k in range(K):
            out_copy(i, b, k).wait()
        if n_steps >= 2:
            for k in range(K):
                out_copy(i - 1, 1 - b, k).wait()


def kernel(x, edit_lens):
    L, D = x.shape
    x_dim, delta_dim = _X_DIM, _DELTA_DIM
    n_edits = edit_lens.shape[0]

    TL = _TL if L % _TL == 0 else L
    K = _K if TL % (_K * 8) == 0 else 1
    n_steps = L // TL

    kern = functools.partial(_gt_kernel, x_dim, _EPS, L, TL, K, n_edits,
                             n_steps)

    xn, dw, db = pl.pallas_call(
        kern,
        out_shape=(
            jax.ShapeDtypeStruct((L, D), jnp.float32),
            jax.ShapeDtypeStruct((x_dim, delta_dim), jnp.float32),
            jax.ShapeDtypeStruct((1, delta_dim), jnp.float32),
        ),
        grid_spec=pltpu.PrefetchScalarGridSpec(
            num_scalar_prefetch=0,
            grid=(n_steps,),
            in_specs=[
                pl.BlockSpec(memory_space=pltpu.MemorySpace.HBM),
                pl.BlockSpec(memory_space=pltpu.MemorySpace.SMEM),
            ],
            out_specs=(
                pl.BlockSpec(memory_space=pltpu.MemorySpace.HBM),
                pl.BlockSpec((x_dim, delta_dim), lambda i: (0, 0)),
                pl.BlockSpec((1, delta_dim), lambda i: (0, 0)),
            ),
            scratch_shapes=[
                pltpu.VMEM((2, TL, D), jnp.float32),
                pltpu.VMEM((2, TL, D), jnp.float32),
                pltpu.SemaphoreType.DMA((2, K)),
                pltpu.SemaphoreType.DMA((2, K)),
            ],
        ),
        compiler_params=pltpu.CompilerParams(
            dimension_semantics=("arbitrary",),
            vmem_limit_bytes=96 << 20,
        ),
    )(x, edit_lens.astype(jnp.int32))

    delta_bias = db[0]
    reps = xn.reshape(1, L, D)
    new_reps = [reps for _ in range(_LAYER_N)]
    return dw, delta_bias, jnp.float32(1.0), new_reps


# speedup vs baseline: 1.1032x; 1.1032x over previous
import functools

import jax
import jax.numpy as jnp
from jax.experimental import pallas as pl
from jax.experimental.pallas import tpu as pltpu

_X_DIM = 512
_DELTA_DIM = 512
_LAYER_N = 2
_EPS = 1e-5
_TL = 2048
_K = 4


def _gt_kernel(x_dim, eps, L, TL, K, n_edits, n_steps,
               x_hbm, len_ref,
               xn_hbm, dw_ref, db_ref,
               x_buf, xn_buf, in_sems, out_sems):
    i = pl.program_id(0)
    b = jax.lax.rem(i, 2)
    CH = TL // K

    def in_copy(step, slot, k):
        return pltpu.make_async_copy(
            x_hbm.at[pl.ds(step * TL + k * CH, CH), :],
            x_buf.at[slot, pl.ds(k * CH, CH), :],
            in_sems.at[slot, k])

    def out_copy(step, slot, k):
        return pltpu.make_async_copy(
            xn_buf.at[slot, pl.ds(k * CH, CH), :],
            xn_hbm.at[pl.ds(step * TL + k * CH, CH), :],
            out_sems.at[slot, k])

    @pl.when(i == 0)
    def _():
        for k in range(K):
            in_copy(0, 0, k).start()
        dw_ref[...] = jnp.zeros_like(dw_ref)
        db_ref[...] = jnp.zeros_like(db_ref)

    @pl.when(i + 1 < n_steps)
    def _():
        for k in range(K):
            in_copy(i + 1, 1 - b, k).start()

    inv_d = jnp.float32(1.0 / x_buf.shape[-1])

    lane_ids = i * TL + jax.lax.broadcasted_iota(jnp.int32, (1, TL), 1)
    scale_lane = jnp.zeros((1, TL), jnp.float32)
    lo = jnp.int32(0)
    for e in range(n_edits):
        ln = len_ref[e]
        hi = lo + ln
        pe = 1.0 / ln.astype(jnp.float32)
        scale_lane = jnp.where((lane_ids >= lo) & (lane_ids < hi),
                               pe, scale_lane)
        lo = hi
    scale_col = scale_lane.reshape(TL, 1) if TL <= 128 else \
        jnp.transpose(scale_lane, (1, 0))

    dw = None
    db = None
    for k in range(K):
        in_copy(i, b, k).wait()

        @pl.when(i >= 2)
        def _():
            out_copy(i - 2, b, k).wait()

        x = x_buf[b, pl.ds(k * CH, CH), :]
        s1 = jnp.sum(x, axis=-1, keepdims=True)
        s2 = jnp.sum(x * x, axis=-1, keepdims=True)
        mean = s1 * inv_d
        var = jnp.maximum(s2 * inv_d - mean * mean, 0.0)
        xn = (x - mean) * jax.lax.rsqrt(var + eps)
        xn_buf[b, pl.ds(k * CH, CH), :] = xn
        out_copy(i, b, k).start()

        scale = scale_col[k * CH:(k + 1) * CH, :]
        x2f = xn[:, x_dim:] * scale
        dbk = jnp.sum(x2f, axis=0, keepdims=True)
        db = dbk if db is None else db + dbk

        x1b = xn[:, :x_dim].astype(jnp.bfloat16)
        x2b = x2f.astype(jnp.bfloat16)
        dwk = jax.lax.dot_general(
            x1b, x2b,
            dimension_numbers=(((0,), (0,)), ((), ())),
            preferred_element_type=jnp.float32)
        dw = dwk if dw is None else dw + dwk

    dw_ref[...] += dw
    db_ref[...] += db

    @pl.when(i == n_steps - 1)
    def _():
        for k in range(K):
            out_copy(i, b, k).wait()
        if n_steps >= 2:
            for k in range(K):
                out_copy(i - 1, 1 - b, k).wait()


def kernel(x, edit_lens):
    L, D = x.shape
    x_dim, delta_dim = _X_DIM, _DELTA_DIM
    n_edits = edit_lens.shape[0]

    TL = _TL if L % _TL == 0 else L
    K = _K if TL % (_K * 8) == 0 else 1
    n_steps = L // TL

    kern = functools.partial(_gt_kernel, x_dim, _EPS, L, TL, K, n_edits,
                             n_steps)

    xn, dw, db = pl.pallas_call(
        kern,
        out_shape=(
            jax.ShapeDtypeStruct((L, D), jnp.float32),
            jax.ShapeDtypeStruct((x_dim, delta_dim), jnp.float32),
            jax.ShapeDtypeStruct((1, delta_dim), jnp.float32),
        ),
        grid_spec=pltpu.PrefetchScalarGridSpec(
            num_scalar_prefetch=0,
            grid=(n_steps,),
            in_specs=[
                pl.BlockSpec(memory_space=pltpu.MemorySpace.HBM),
                pl.BlockSpec(memory_space=pltpu.MemorySpace.SMEM),
            ],
            out_specs=(
                pl.BlockSpec(memory_space=pltpu.MemorySpace.HBM),
                pl.BlockSpec((x_dim, delta_dim), lambda i: (0, 0)),
                pl.BlockSpec((1, delta_dim), lambda i: (0, 0)),
            ),
            scratch_shapes=[
                pltpu.VMEM((2, TL, D), jnp.float32),
                pltpu.VMEM((2, TL, D), jnp.float32),
                pltpu.SemaphoreType.DMA((2, K)),
                pltpu.SemaphoreType.DMA((2, K)),
            ],
        ),
        compiler_params=pltpu.CompilerParams(
            dimension_semantics=("arbitrary",),
            vmem_limit_bytes=96 << 20,
        ),
    )(x, edit_lens.astype(jnp.int32))

    delta_bias = db[0]
    reps = xn.reshape(1, L, D)
    new_reps = [reps for _ in range(_LAYER_N)]
    return dw, delta_bias, jnp.float32(1.0), new_reps
